# baseline (speedup 1.0000x reference)
import functools

import jax
import jax.numpy as jnp
from jax import lax
from jax.experimental import pallas as pl
from jax.experimental.pallas import tpu as pltpu
from jax.experimental.pallas import tpu_sc as plsc

N_NODES = 10000
D = 128
N_EDGES = 320000
NC = 2
NS = 16
NW = NC * NS
CHUNK = 128
NCHUNK = 79
EDGES_PER_W = NCHUNK * CHUNK
E_PAD = NW * EDGES_PER_W
N_PAD = 10240
ROWS_PER_TILE = N_PAD // NS
DUMP_ROW = 10176
RING = 8
IDX_AHEAD = 6

_MESH = plsc.VectorSubcoreMesh(core_axis_name="c", subcore_axis_name="s")


@functools.partial(
    pl.kernel,
    out_type=jax.ShapeDtypeStruct((NW, N_PAD), jnp.float32),
    mesh=_MESH,
    scratch_types=[
        pltpu.VMEM((NCHUNK, CHUNK), jnp.int32),
        pltpu.VMEM((N_PAD,), jnp.float32),
    ],
    compiler_params=pltpu.CompilerParams(needs_layout_passes=False),
)
def _deg_kernel(dst_hbm, out_hbm, idx_d, degp):
    c = lax.axis_index("c")
    s = lax.axis_index("s")
    w = c * NS + s

    def zero(i, carry):
        degp[pl.ds(i * 16, 16)] = jnp.zeros((16,), jnp.float32)
        return carry

    lax.fori_loop(0, N_PAD // 16, zero, 0)

    pltpu.sync_copy(dst_hbm.at[w], idx_d)
    ones = jnp.ones((16,), jnp.float32)

    def body(j, carry):
        for k in range(CHUNK // 16):
            idx16 = idx_d[j, pl.ds(k * 16, 16)]
            plsc.addupdate_scatter(degp, [idx16], ones)
        return carry

    lax.fori_loop(0, NCHUNK, body, 0)
    pltpu.sync_copy(degp, out_hbm.at[w])


@functools.partial(
    pl.kernel,
    out_type=jax.ShapeDtypeStruct((NC, N_PAD, D), jnp.float32),
    mesh=_MESH,
    scratch_types=[
        pltpu.VMEM((RING, CHUNK), jnp.int32),
        pltpu.VMEM((RING, CHUNK), jnp.int32),
        pltpu.VMEM((2 * CHUNK, D), jnp.float32),
        pltpu.VMEM_SHARED((N_PAD, D), jnp.float32),
        pltpu.SemaphoreType.DMA,
        pltpu.SemaphoreType.DMA,
        pltpu.SemaphoreType.DMA,
    ],
)
def _scatter_kernel(src_hbm, dst_hbm, g_hbm, out_hbm,
                    s_ring, d_ring, rows, acc, sem_s, sem_d, sem_g):
    c = lax.axis_index("c")
    s = lax.axis_index("s")
    w = c * NS + s

    def zrow(i, carry):
        def zcol(k, inner):
            rows[i, pl.ds(k * 16, 16)] = jnp.zeros((16,), jnp.float32)
            return inner
        return lax.fori_loop(0, D // 16, zcol, carry)

    lax.fori_loop(0, CHUNK, zrow, 0)
    for z in range(ROWS_PER_TILE // CHUNK):
        pltpu.sync_copy(
            rows.at[pl.ds(0, CHUNK)],
            acc.at[pl.ds(s * ROWS_PER_TILE + z * CHUNK, CHUNK)])
    plsc.subcore_barrier()

    def fire_idx(j):
        pltpu.async_copy(src_hbm.at[w, j], s_ring.at[lax.rem(j, RING)], sem_s)
        pltpu.async_copy(dst_hbm.at[w, j], d_ring.at[lax.rem(j, RING)], sem_d)

    def wait_idx():
        pltpu.make_async_copy(src_hbm.at[w, 0], s_ring.at[0], sem_s).wait()
        pltpu.make_async_copy(dst_hbm.at[w, 0], d_ring.at[0], sem_d).wait()

    def rows_slot(j):
        return rows.at[pl.ds(pl.multiple_of(lax.rem(j, 2) * CHUNK, CHUNK),
                             CHUNK)]

    def fire_gather(j):
        pltpu.async_copy(g_hbm.at[s_ring.at[lax.rem(j, RING)]],
                         rows_slot(j), sem_g)

    for j in range(IDX_AHEAD):
        fire_idx(j)
    for _ in range(3):
        wait_idx()
    fire_gather(0)
    fire_gather(1)

    def body(j, carry):
        @pl.when(j < NCHUNK - 3)
        def _():
            wait_idx()

        pltpu.make_async_copy(g_hbm.at[s_ring.at[0]], rows_slot(j),
                              sem_g).wait()
        pltpu.sync_copy(rows_slot(j), acc.at[d_ring.at[lax.rem(j, RING)]],
                        add=True)

        @pl.when(j + 2 < NCHUNK)
        def _():
            fire_gather(j + 2)

        @pl.when(j + IDX_AHEAD < NCHUNK)
        def _():
            fire_idx(j + IDX_AHEAD)

        return carry

    lax.fori_loop(0, NCHUNK, body, 0)
    plsc.subcore_barrier()
    pltpu.sync_copy(acc.at[pl.ds(s * ROWS_PER_TILE, ROWS_PER_TILE)],
                    out_hbm.at[c, pl.ds(s * ROWS_PER_TILE, ROWS_PER_TILE)])


def _tc1_body(degpt_ref, x_ref, w1_ref, dis_ref, g1_ref):
    deg = jnp.sum(degpt_ref[...], axis=1, keepdims=True)[:N_NODES] + 1.0
    dis = lax.rsqrt(deg)
    dis_ref[...] = dis
    h = jnp.dot(x_ref[...], w1_ref[...],
                preferred_element_type=jnp.float32,
                precision=lax.Precision.HIGHEST)
    g1_ref[...] = h * dis


def _tc2_body(s1_ref, g1_ref, dis_ref, b1_ref, w2_ref, g2_ref):
    dis = dis_ref[...]
    ssum = (s1_ref[0] + s1_ref[1])[:N_NODES]
    t = (ssum + g1_ref[...]) * dis + b1_ref[...]
    t = jnp.maximum(t, 0.0)
    h2 = jnp.dot(t, w2_ref[...],
                 preferred_element_type=jnp.float32,
                 precision=lax.Precision.HIGHEST)
    g2_ref[...] = h2 * dis


def _tc3_body(s2_ref, g2_ref, dis_ref, b2_ref, out_ref):
    ssum = (s2_ref[0] + s2_ref[1])[:N_NODES]
    out_ref[...] = (ssum + g2_ref[...]) * dis_ref[...] + b2_ref[...]


_tc1 = pl.pallas_call(
    _tc1_body,
    out_shape=[jax.ShapeDtypeStruct((N_NODES, 1), jnp.float32),
               jax.ShapeDtypeStruct((N_NODES, D), jnp.float32)],
)

_tc2 = pl.pallas_call(
    _tc2_body,
    out_shape=jax.ShapeDtypeStruct((N_NODES, D), jnp.float32),
)

_tc3 = pl.pallas_call(
    _tc3_body,
    out_shape=jax.ShapeDtypeStruct((N_NODES, D), jnp.float32),
)


def kernel(x, edge_index, W1, b1, W2, b2):
    pad = E_PAD - N_EDGES
    src = jnp.concatenate(
        [edge_index[0].astype(jnp.int32), jnp.zeros((pad,), jnp.int32)]
    ).reshape(NW, NCHUNK, CHUNK)
    dst = jnp.concatenate(
        [edge_index[1].astype(jnp.int32),
         jnp.full((pad,), DUMP_ROW, jnp.int32)]
    ).reshape(NW, NCHUNK, CHUNK)
    b1r = b1.reshape(1, D)
    b2r = b2.reshape(1, D)

    degp = _deg_kernel(dst)
    dis, g1 = _tc1(degp.T, x, W1)
    s1 = _scatter_kernel(src, dst, g1)
    g2 = _tc2(s1, g1, dis, b1r, W2)
    s2 = _scatter_kernel(src, dst, g2)
    return _tc3(s2, g2, dis, b2r)


# speedup vs baseline: 2.0752x; 2.0752x over previous
import functools

import jax
import jax.numpy as jnp
from jax import lax
from jax.experimental import pallas as pl
from jax.experimental.pallas import tpu as pltpu
from jax.experimental.pallas import tpu_sc as plsc

N_NODES = 10000
D = 128
N_EDGES = 320000
NC = 2
NS = 16
NW = NC * NS
CHUNK = 128
NCHUNK = 79
EDGES_PER_W = NCHUNK * CHUNK
E_PAD = NW * EDGES_PER_W
N_PAD = 10240
ROWS_PER_TILE = N_PAD // NS
RING = 8
IDX_AHEAD = 6

_MESH = plsc.VectorSubcoreMesh(core_axis_name="c", subcore_axis_name="s")


@functools.partial(
    pl.kernel,
    out_type=jax.ShapeDtypeStruct((NW, N_PAD), jnp.float32),
    mesh=_MESH,
    scratch_types=[
        pltpu.VMEM((NCHUNK, CHUNK), jnp.int32),
        pltpu.VMEM((N_PAD,), jnp.float32),
    ],
    compiler_params=pltpu.CompilerParams(needs_layout_passes=False),
)
def _deg_kernel(dst_hbm, out_hbm, idx_d, degp):
    c = lax.axis_index("c")
    s = lax.axis_index("s")
    w = c * NS + s

    def zero(i, carry):
        degp[pl.ds(i * 16, 16)] = jnp.zeros((16,), jnp.float32)
        return carry

    lax.fori_loop(0, N_PAD // 16, zero, 0)

    pltpu.sync_copy(dst_hbm.at[w], idx_d)
    ones = jnp.ones((16,), jnp.float32)

    def body(j, carry):
        for k in range(CHUNK // 16):
            idx16 = idx_d[j, pl.ds(k * 16, 16)]
            plsc.addupdate_scatter(degp, [idx16], ones)
        return carry

    lax.fori_loop(0, NCHUNK, body, 0)
    pltpu.sync_copy(degp, out_hbm.at[w])


@functools.partial(
    pl.kernel,
    out_type=jax.ShapeDtypeStruct((NC, N_PAD, D), jnp.float32),
    mesh=_MESH,
    scratch_types=[
        pltpu.VMEM((RING, CHUNK), jnp.int32),
        pltpu.VMEM((RING, CHUNK), jnp.int32),
        pltpu.VMEM((2 * CHUNK, D), jnp.float32),
        pltpu.VMEM_SHARED((N_PAD, D), jnp.float32),
        pltpu.SemaphoreType.DMA,
        pltpu.SemaphoreType.DMA,
        pltpu.SemaphoreType.DMA,
    ],
)
def _scatter_kernel(src_hbm, dst_hbm, g_hbm, out_hbm,
                    s_ring, d_ring, rows, acc, sem_s, sem_d, sem_g):
    c = lax.axis_index("c")
    s = lax.axis_index("s")
    w = c * NS + s

    def zrow(i, carry):
        def zcol(k, inner):
            rows[i, pl.ds(k * 16, 16)] = jnp.zeros((16,), jnp.float32)
            return inner
        return lax.fori_loop(0, D // 16, zcol, carry)

    lax.fori_loop(0, CHUNK, zrow, 0)
    for z in range(ROWS_PER_TILE // CHUNK):
        pltpu.sync_copy(
            rows.at[pl.ds(0, CHUNK)],
            acc.at[pl.ds(s * ROWS_PER_TILE + z * CHUNK, CHUNK)])
    plsc.subcore_barrier()

    def fire_idx(j):
        pltpu.async_copy(src_hbm.at[w, j], s_ring.at[lax.rem(j, RING)], sem_s)
        pltpu.async_copy(dst_hbm.at[w, j], d_ring.at[lax.rem(j, RING)], sem_d)

    def wait_idx():
        pltpu.make_async_copy(src_hbm.at[w, 0], s_ring.at[0], sem_s).wait()
        pltpu.make_async_copy(dst_hbm.at[w, 0], d_ring.at[0], sem_d).wait()

    def rows_slot(j):
        return rows.at[pl.ds(pl.multiple_of(lax.rem(j, 2) * CHUNK, CHUNK),
                             CHUNK)]

    def fire_gather(j):
        pltpu.async_copy(g_hbm.at[s_ring.at[lax.rem(j, RING)]],
                         rows_slot(j), sem_g)

    for j in range(IDX_AHEAD):
        fire_idx(j)
    for _ in range(3):
        wait_idx()
    fire_gather(0)
    fire_gather(1)

    def body(j, carry):
        @pl.when(j < NCHUNK - 3)
        def _():
            wait_idx()

        pltpu.make_async_copy(g_hbm.at[s_ring.at[0]], rows_slot(j),
                              sem_g).wait()
        pltpu.sync_copy(rows_slot(j), acc.at[d_ring.at[lax.rem(j, RING)]],
                        add=True)

        @pl.when(j + 2 < NCHUNK)
        def _():
            fire_gather(j + 2)

        @pl.when(j + IDX_AHEAD < NCHUNK)
        def _():
            fire_idx(j + IDX_AHEAD)

        return carry

    lax.fori_loop(0, NCHUNK, body, 0)
    plsc.subcore_barrier()
    pltpu.sync_copy(acc.at[pl.ds(s * ROWS_PER_TILE, ROWS_PER_TILE)],
                    out_hbm.at[c, pl.ds(s * ROWS_PER_TILE, ROWS_PER_TILE)])


def _tc1_body(degpt_ref, x_ref, w1_ref, dis_ref, g1_ref):
    deg = jnp.sum(degpt_ref[...], axis=1, keepdims=True)[:N_NODES] + 1.0
    dis = lax.rsqrt(deg)
    dis_ref[...] = dis
    h = jnp.dot(x_ref[...], w1_ref[...],
                preferred_element_type=jnp.float32,
                precision=lax.Precision.HIGHEST)
    g1_ref[...] = h * dis


def _tc2_body(s1_ref, g1_ref, dis_ref, b1_ref, w2_ref, g2_ref):
    dis = dis_ref[...]
    ssum = (s1_ref[0] + s1_ref[1])[:N_NODES]
    t = (ssum + g1_ref[...]) * dis + b1_ref[...]
    t = jnp.maximum(t, 0.0)
    h2 = jnp.dot(t, w2_ref[...],
                 preferred_element_type=jnp.float32,
                 precision=lax.Precision.HIGHEST)
    g2_ref[...] = h2 * dis


def _tc3_body(s2_ref, g2_ref, dis_ref, b2_ref, out_ref):
    ssum = (s2_ref[0] + s2_ref[1])[:N_NODES]
    out_ref[...] = (ssum + g2_ref[...]) * dis_ref[...] + b2_ref[...]


_tc1 = pl.pallas_call(
    _tc1_body,
    out_shape=[jax.ShapeDtypeStruct((N_NODES, 1), jnp.float32),
               jax.ShapeDtypeStruct((N_NODES, D), jnp.float32)],
)

_tc2 = pl.pallas_call(
    _tc2_body,
    out_shape=jax.ShapeDtypeStruct((N_NODES, D), jnp.float32),
)

_tc3 = pl.pallas_call(
    _tc3_body,
    out_shape=jax.ShapeDtypeStruct((N_NODES, D), jnp.float32),
)


def kernel(x, edge_index, W1, b1, W2, b2):
    pad_w = EDGES_PER_W - N_EDGES // NW
    pad_src = jnp.broadcast_to(jnp.arange(pad_w, dtype=jnp.int32),
                               (NW, pad_w))
    pad_dst = jnp.broadcast_to(
        N_NODES + jnp.arange(pad_w, dtype=jnp.int32), (NW, pad_w))
    src = jnp.concatenate(
        [edge_index[0].astype(jnp.int32).reshape(NW, N_EDGES // NW),
         pad_src], axis=1).reshape(NW, NCHUNK, CHUNK)
    dst = jnp.concatenate(
        [edge_index[1].astype(jnp.int32).reshape(NW, N_EDGES // NW),
         pad_dst], axis=1).reshape(NW, NCHUNK, CHUNK)
    b1r = b1.reshape(1, D)
    b2r = b2.reshape(1, D)

    degp = _deg_kernel(dst)
    dis, g1 = _tc1(degp.T, x, W1)
    s1 = _scatter_kernel(src, dst, g1)
    g2 = _tc2(s1, g1, dis, b1r, W2)
    s2 = _scatter_kernel(src, dst, g2)
    return _tc3(s2, g2, dis, b2r)


# speedup vs baseline: 2.0850x; 1.0048x over previous
import functools

import jax
import jax.numpy as jnp
from jax import lax
from jax.experimental import pallas as pl
from jax.experimental.pallas import tpu as pltpu
from jax.experimental.pallas import tpu_sc as plsc

N_NODES = 10000
D = 128
N_EDGES = 320000
NC = 2
NS = 16
NW = NC * NS
CHUNK = 128
NCHUNK = 79
EDGES_PER_W = NCHUNK * CHUNK
E_PAD = NW * EDGES_PER_W
N_PAD = 10240
ROWS_PER_TILE = N_PAD // NS
RING = 8
IDX_AHEAD = 6

_MESH = plsc.VectorSubcoreMesh(core_axis_name="c", subcore_axis_name="s")


@functools.partial(
    pl.kernel,
    out_type=jax.ShapeDtypeStruct((NW, N_PAD), jnp.float32),
    mesh=_MESH,
    scratch_types=[
        pltpu.VMEM((NCHUNK, CHUNK), jnp.int32),
        pltpu.VMEM((N_PAD,), jnp.float32),
    ],
    compiler_params=pltpu.CompilerParams(needs_layout_passes=False),
)
def _deg_kernel(dst_hbm, out_hbm, idx_d, degp):
    c = lax.axis_index("c")
    s = lax.axis_index("s")
    w = c * NS + s

    def zero(i, carry):
        degp[pl.ds(i * 16, 16)] = jnp.zeros((16,), jnp.float32)
        return carry

    lax.fori_loop(0, N_PAD // 16, zero, 0)

    pltpu.sync_copy(dst_hbm.at[w], idx_d)
    ones = jnp.ones((16,), jnp.float32)

    def body(j, carry):
        for k in range(CHUNK // 16):
            idx16 = idx_d[j, pl.ds(k * 16, 16)]
            plsc.addupdate_scatter(degp, [idx16], ones)
        return carry

    lax.fori_loop(0, NCHUNK, body, 0)
    pltpu.sync_copy(degp, out_hbm.at[w])


@functools.partial(
    pl.kernel,
    out_type=jax.ShapeDtypeStruct((NC, N_PAD, D), jnp.float32),
    mesh=_MESH,
    scratch_types=[
        pltpu.VMEM((RING, CHUNK), jnp.int32),
        pltpu.VMEM((RING, CHUNK), jnp.int32),
        pltpu.VMEM((2 * CHUNK, D), jnp.float32),
        pltpu.VMEM_SHARED((N_PAD, D), jnp.float32),
        pltpu.SemaphoreType.DMA,
        pltpu.SemaphoreType.DMA,
        pltpu.SemaphoreType.DMA,
    ],
)
def _scatter_kernel(src_hbm, dst_hbm, g_hbm, out_hbm,
                    s_ring, d_ring, rows, acc, sem_s, sem_d, sem_g):
    c = lax.axis_index("c")
    s = lax.axis_index("s")
    w = c * NS + s

    def fire_idx(j):
        pltpu.async_copy(src_hbm.at[w, j], s_ring.at[j & (RING - 1)], sem_s)
        pltpu.async_copy(dst_hbm.at[w, j], d_ring.at[j & (RING - 1)], sem_d)

    def wait_idx():
        pltpu.make_async_copy(src_hbm.at[w, 0], s_ring.at[0], sem_s).wait()
        pltpu.make_async_copy(dst_hbm.at[w, 0], d_ring.at[0], sem_d).wait()

    def rows_slot(j):
        return rows.at[pl.ds(pl.multiple_of((j & 1) * CHUNK, CHUNK), CHUNK)]

    def fire_gather(j):
        pltpu.async_copy(g_hbm.at[s_ring.at[j & (RING - 1)]],
                         rows_slot(j), sem_g)

    for j in range(IDX_AHEAD):
        fire_idx(j)

    zero16 = jnp.zeros((16,), jnp.float32)

    def zrow(i, carry):
        for k in range(D // 16):
            rows[i, pl.ds(k * 16, 16)] = zero16
        return carry

    lax.fori_loop(0, CHUNK, zrow, 0)
    for z in range(ROWS_PER_TILE // CHUNK):
        pltpu.async_copy(
            rows.at[pl.ds(0, CHUNK)],
            acc.at[pl.ds(s * ROWS_PER_TILE + z * CHUNK, CHUNK)], sem_g)
    for _ in range(3):
        wait_idx()
    for z in range(ROWS_PER_TILE // CHUNK):
        pltpu.make_async_copy(
            rows.at[pl.ds(0, CHUNK)],
            acc.at[pl.ds(0, CHUNK)], sem_g).wait()
    fire_gather(0)
    fire_gather(1)
    plsc.subcore_barrier()

    def body(j, carry):
        @pl.when(j < NCHUNK - 3)
        def _():
            wait_idx()

        pltpu.make_async_copy(g_hbm.at[s_ring.at[0]], rows_slot(j),
                              sem_g).wait()
        pltpu.sync_copy(rows_slot(j), acc.at[d_ring.at[lax.rem(j, RING)]],
                        add=True)

        @pl.when(j + 2 < NCHUNK)
        def _():
            fire_gather(j + 2)

        @pl.when(j + IDX_AHEAD < NCHUNK)
        def _():
            fire_idx(j + IDX_AHEAD)

        return carry

    lax.fori_loop(0, NCHUNK, body, 0)
    plsc.subcore_barrier()
    pltpu.sync_copy(acc.at[pl.ds(s * ROWS_PER_TILE, ROWS_PER_TILE)],
                    out_hbm.at[c, pl.ds(s * ROWS_PER_TILE, ROWS_PER_TILE)])


def _tc1_body(degpt_ref, x_ref, w1_ref, dis_ref, g1_ref):
    deg = jnp.sum(degpt_ref[...], axis=1, keepdims=True)[:N_NODES] + 1.0
    dis = lax.rsqrt(deg)
    dis_ref[...] = dis
    h = jnp.dot(x_ref[...], w1_ref[...],
                preferred_element_type=jnp.float32,
                precision=lax.Precision.HIGHEST)
    g1_ref[...] = h * dis


def _tc2_body(s1_ref, g1_ref, dis_ref, b1_ref, w2_ref, g2_ref):
    dis = dis_ref[...]
    ssum = (s1_ref[0] + s1_ref[1])[:N_NODES]
    t = (ssum + g1_ref[...]) * dis + b1_ref[...]
    t = jnp.maximum(t, 0.0)
    h2 = jnp.dot(t, w2_ref[...],
                 preferred_element_type=jnp.float32,
                 precision=lax.Precision.HIGHEST)
    g2_ref[...] = h2 * dis


def _tc3_body(s2_ref, g2_ref, dis_ref, b2_ref, out_ref):
    ssum = (s2_ref[0] + s2_ref[1])[:N_NODES]
    out_ref[...] = (ssum + g2_ref[...]) * dis_ref[...] + b2_ref[...]


_tc1 = pl.pallas_call(
    _tc1_body,
    out_shape=[jax.ShapeDtypeStruct((N_NODES, 1), jnp.float32),
               jax.ShapeDtypeStruct((N_NODES, D), jnp.float32)],
)

_tc2 = pl.pallas_call(
    _tc2_body,
    out_shape=jax.ShapeDtypeStruct((N_NODES, D), jnp.float32),
)

_tc3 = pl.pallas_call(
    _tc3_body,
    out_shape=jax.ShapeDtypeStruct((N_NODES, D), jnp.float32),
)


def kernel(x, edge_index, W1, b1, W2, b2):
    pad_w = EDGES_PER_W - N_EDGES // NW
    pad_src = jnp.broadcast_to(jnp.arange(pad_w, dtype=jnp.int32),
                               (NW, pad_w))
    pad_dst = jnp.broadcast_to(
        N_NODES + jnp.arange(pad_w, dtype=jnp.int32), (NW, pad_w))
    src = jnp.concatenate(
        [edge_index[0].astype(jnp.int32).reshape(NW, N_EDGES // NW),
         pad_src], axis=1).reshape(NW, NCHUNK, CHUNK)
    dst = jnp.concatenate(
        [edge_index[1].astype(jnp.int32).reshape(NW, N_EDGES // NW),
         pad_dst], axis=1).reshape(NW, NCHUNK, CHUNK)
    b1r = b1.reshape(1, D)
    b2r = b2.reshape(1, D)

    degp = _deg_kernel(dst)
    dis, g1 = _tc1(degp.T, x, W1)
    s1 = _scatter_kernel(src, dst, g1)
    g2 = _tc2(s1, g1, dis, b1r, W2)
    s2 = _scatter_kernel(src, dst, g2)
    return _tc3(s2, g2, dis, b2r)


# speedup vs baseline: 2.1192x; 1.0164x over previous
import functools

import jax
import jax.numpy as jnp
from jax import lax
from jax.experimental import pallas as pl
from jax.experimental.pallas import tpu as pltpu
from jax.experimental.pallas import tpu_sc as plsc

N_NODES = 10000
D = 128
N_EDGES = 320000
NC = 2
NS = 16
NW = NC * NS
EPW = N_EDGES // NW
CHUNK = 128
NCHUNK = 79
OVER = NCHUNK * CHUNK - EPW
N_PAD = 10240
ROWS_PER_TILE = N_PAD // NS
RING = 8
IDX_AHEAD = 6

_MESH = plsc.VectorSubcoreMesh(core_axis_name="c", subcore_axis_name="s")


@functools.partial(
    pl.kernel,
    out_type=jax.ShapeDtypeStruct((NW, N_NODES), jnp.float32),
    mesh=_MESH,
    scratch_types=[
        pltpu.VMEM((EPW,), jnp.int32),
        pltpu.VMEM((N_NODES,), jnp.float32),
    ],
    compiler_params=pltpu.CompilerParams(needs_layout_passes=False),
)
def _deg_kernel(dst_hbm, out_hbm, idx_d, degp):
    c = lax.axis_index("c")
    s = lax.axis_index("s")
    w = c * NS + s

    def zero(i, carry):
        degp[pl.ds(i * 16, 16)] = jnp.zeros((16,), jnp.float32)
        return carry

    lax.fori_loop(0, N_NODES // 16, zero, 0)

    pltpu.sync_copy(dst_hbm.at[pl.ds(w * EPW, EPW)], idx_d)
    ones = jnp.ones((16,), jnp.float32)

    def body(i, carry):
        idx16 = idx_d[pl.ds(i * 16, 16)]
        plsc.addupdate_scatter(degp, [idx16], ones)
        return carry

    lax.fori_loop(0, EPW // 16, body, 0)
    pltpu.sync_copy(degp, out_hbm.at[w])


@functools.partial(
    pl.kernel,
    out_type=jax.ShapeDtypeStruct((NC, N_PAD, D), jnp.float32),
    mesh=_MESH,
    scratch_types=[
        pltpu.VMEM((RING, CHUNK), jnp.int32),
        pltpu.VMEM((RING, CHUNK), jnp.int32),
        pltpu.VMEM((2 * CHUNK, D), jnp.float32),
        pltpu.VMEM_SHARED((N_PAD, D), jnp.float32),
        pltpu.SemaphoreType.DMA,
        pltpu.SemaphoreType.DMA,
        pltpu.SemaphoreType.DMA,
    ],
)
def _scatter_kernel(src_hbm, dst_hbm, g_hbm, out_hbm,
                    s_ring, d_ring, rows, acc, sem_s, sem_d, sem_g):
    c = lax.axis_index("c")
    s = lax.axis_index("s")
    w = c * NS + s
    base = w * EPW

    def fire_idx(j):
        k = j & (RING - 1)
        pltpu.async_copy(src_hbm.at[pl.ds(base + j * CHUNK, CHUNK)],
                         s_ring.at[k], sem_s)
        pltpu.async_copy(dst_hbm.at[pl.ds(base + j * CHUNK, CHUNK)],
                         d_ring.at[k], sem_d)

    def wait_idx():
        pltpu.make_async_copy(src_hbm.at[pl.ds(0, CHUNK)],
                              s_ring.at[0], sem_s).wait()
        pltpu.make_async_copy(dst_hbm.at[pl.ds(0, CHUNK)],
                              d_ring.at[0], sem_d).wait()

    def rows_slot(j):
        return rows.at[pl.ds(pl.multiple_of((j & 1) * CHUNK, CHUNK), CHUNK)]

    def fire_gather(j):
        pltpu.async_copy(g_hbm.at[s_ring.at[j & (RING - 1)]],
                         rows_slot(j), sem_g)

    for j in range(IDX_AHEAD):
        fire_idx(j)

    zero16 = jnp.zeros((16,), jnp.float32)

    def zrow(i, carry):
        for k in range(D // 16):
            rows[i, pl.ds(k * 16, 16)] = zero16
        return carry

    lax.fori_loop(0, CHUNK, zrow, 0)
    for z in range(ROWS_PER_TILE // CHUNK):
        pltpu.async_copy(
            rows.at[pl.ds(0, CHUNK)],
            acc.at[pl.ds(s * ROWS_PER_TILE + z * CHUNK, CHUNK)], sem_g)
    for _ in range(3):
        wait_idx()
    for z in range(ROWS_PER_TILE // CHUNK):
        pltpu.make_async_copy(
            rows.at[pl.ds(0, CHUNK)],
            acc.at[pl.ds(0, CHUNK)], sem_g).wait()
    fire_gather(0)
    fire_gather(1)
    plsc.subcore_barrier()

    def body(j, carry):
        @pl.when(j < NCHUNK - 3)
        def _():
            wait_idx()

        pltpu.make_async_copy(g_hbm.at[s_ring.at[0]], rows_slot(j),
                              sem_g).wait()

        @pl.when(j == NCHUNK - 1)
        def _():
            k = j & (RING - 1)
            for t in range(OVER // 16):
                d_ring[k, pl.ds(CHUNK - OVER + t * 16, 16)] = (
                    jnp.arange(16, dtype=jnp.int32) + (N_NODES + t * 16))

        pltpu.sync_copy(rows_slot(j), acc.at[d_ring.at[j & (RING - 1)]],
                        add=True)

        @pl.when(j + 2 < NCHUNK)
        def _():
            fire_gather(j + 2)

        @pl.when(j + IDX_AHEAD < NCHUNK)
        def _():
            fire_idx(j + IDX_AHEAD)

        return carry

    lax.fori_loop(0, NCHUNK, body, 0)
    plsc.subcore_barrier()
    pltpu.sync_copy(acc.at[pl.ds(s * ROWS_PER_TILE, ROWS_PER_TILE)],
                    out_hbm.at[c, pl.ds(s * ROWS_PER_TILE, ROWS_PER_TILE)])


def _tc1_body(degpt_ref, x_ref, w1_ref, dis_ref, g1_ref):
    deg = jnp.sum(degpt_ref[...], axis=1, keepdims=True) + 1.0
    dis = lax.rsqrt(deg)
    dis_ref[...] = dis
    h = jnp.dot(x_ref[...], w1_ref[...],
                preferred_element_type=jnp.float32,
                precision=lax.Precision.DEFAULT)
    g1_ref[...] = h * dis


def _tc2_body(s1_ref, g1_ref, dis_ref, b1_ref, w2_ref, g2_ref):
    dis = dis_ref[...]
    ssum = (s1_ref[0] + s1_ref[1])[:N_NODES]
    t = (ssum + g1_ref[...]) * dis + b1_ref[...]
    t = jnp.maximum(t, 0.0)
    h2 = jnp.dot(t, w2_ref[...],
                 preferred_element_type=jnp.float32,
                 precision=lax.Precision.DEFAULT)
    g2_ref[...] = h2 * dis


def _tc3_body(s2_ref, g2_ref, dis_ref, b2_ref, out_ref):
    ssum = (s2_ref[0] + s2_ref[1])[:N_NODES]
    out_ref[...] = (ssum + g2_ref[...]) * dis_ref[...] + b2_ref[...]


_tc1 = pl.pallas_call(
    _tc1_body,
    out_shape=[jax.ShapeDtypeStruct((N_NODES, 1), jnp.float32),
               jax.ShapeDtypeStruct((N_NODES, D), jnp.float32)],
)

_tc2 = pl.pallas_call(
    _tc2_body,
    out_shape=jax.ShapeDtypeStruct((N_NODES, D), jnp.float32),
)

_tc3 = pl.pallas_call(
    _tc3_body,
    out_shape=jax.ShapeDtypeStruct((N_NODES, D), jnp.float32),
)


def kernel(x, edge_index, W1, b1, W2, b2):
    pad = jnp.zeros((OVER,), jnp.int32)
    src = jnp.concatenate([edge_index[0].astype(jnp.int32), pad])
    dst = jnp.concatenate([edge_index[1].astype(jnp.int32), pad])
    b1r = b1.reshape(1, D)
    b2r = b2.reshape(1, D)

    degp = _deg_kernel(dst)
    dis, g1 = _tc1(degp.T, x, W1)
    s1 = _scatter_kernel(src, dst, g1)
    g2 = _tc2(s1, g1, dis, b1r, W2)
    s2 = _scatter_kernel(src, dst, g2)
    return _tc3(s2, g2, dis, b2r)


# speedup vs baseline: 2.2369x; 1.0556x over previous
import functools

import jax
import jax.numpy as jnp
from jax import lax
from jax.experimental import pallas as pl
from jax.experimental.pallas import tpu as pltpu
from jax.experimental.pallas import tpu_sc as plsc

N_NODES = 10000
D = 128
N_EDGES = 320000
NC = 2
NS = 16
NW = NC * NS
CHUNK = 128
TOTAL_CHUNKS = N_EDGES // CHUNK
BASE_CHUNKS = TOTAL_CHUNKS // NW
EXTRA = TOTAL_CHUNKS - NW * BASE_CHUNKS
NCHUNK = BASE_CHUNKS + 1
N_PAD = 10240
ROWS_PER_TILE = N_PAD // NS
RING = 8
IDX_AHEAD = 6

_MESH = plsc.VectorSubcoreMesh(core_axis_name="c", subcore_axis_name="s")


@functools.partial(
    pl.kernel,
    out_type=jax.ShapeDtypeStruct((NW, N_NODES), jnp.float32),
    mesh=_MESH,
    scratch_types=[
        pltpu.VMEM((2, NCHUNK * CHUNK), jnp.int32),
        pltpu.VMEM((N_NODES,), jnp.float32),
    ],
    compiler_params=pltpu.CompilerParams(needs_layout_passes=False),
)
def _deg_kernel(edges_hbm, out_hbm, idx_d, degp):
    c = lax.axis_index("c")
    s = lax.axis_index("s")
    w = c * NS + s

    def zero(i, carry):
        degp[pl.ds(i * 16, 16)] = jnp.zeros((16,), jnp.float32)
        return carry

    lax.fori_loop(0, N_NODES // 16, zero, 0)

    c0 = w * BASE_CHUNKS + jnp.minimum(w, EXTRA)
    full = w < EXTRA
    off = pl.multiple_of(c0 * CHUNK, CHUNK)

    @pl.when(full)
    def _():
        pltpu.sync_copy(
            edges_hbm.at[pl.ds(0, 2), pl.ds(off, NCHUNK * CHUNK)], idx_d)

    @pl.when(jnp.logical_not(full))
    def _():
        pltpu.sync_copy(
            edges_hbm.at[pl.ds(0, 2), pl.ds(off, BASE_CHUNKS * CHUNK)],
            idx_d.at[pl.ds(0, 2), pl.ds(0, BASE_CHUNKS * CHUNK)])

    ones = jnp.ones((16,), jnp.float32)

    def body(i, carry):
        idx16 = idx_d[1, pl.ds(i * 16, 16)]
        plsc.addupdate_scatter(degp, [idx16], ones)
        return carry

    lax.fori_loop(0, BASE_CHUNKS * CHUNK // 16, body, 0)

    @pl.when(full)
    def _():
        lax.fori_loop(BASE_CHUNKS * CHUNK // 16, NCHUNK * CHUNK // 16,
                      body, 0)

    pltpu.sync_copy(degp, out_hbm.at[w])


@functools.partial(
    pl.kernel,
    out_type=jax.ShapeDtypeStruct((NC, N_PAD, D), jnp.float32),
    mesh=_MESH,
    scratch_types=[
        pltpu.VMEM((RING, 2, CHUNK), jnp.int32),
        pltpu.VMEM((2 * CHUNK, D), jnp.float32),
        pltpu.VMEM_SHARED((N_PAD, D), jnp.float32),
        pltpu.SemaphoreType.DMA,
        pltpu.SemaphoreType.DMA,
    ],
)
def _scatter_kernel(edges_hbm, g_hbm, out_hbm,
                    ring, rows, acc, sem_i, sem_g):
    c = lax.axis_index("c")
    s = lax.axis_index("s")
    w = c * NS + s
    c0 = w * BASE_CHUNKS + jnp.minimum(w, EXTRA)
    full = w < EXTRA

    def fire_idx(j):
        k = j & (RING - 1)
        g = jnp.minimum(c0 + j, TOTAL_CHUNKS - 1)
        off = pl.multiple_of(g * CHUNK, CHUNK)
        pltpu.async_copy(edges_hbm.at[pl.ds(0, 2), pl.ds(off, CHUNK)],
                         ring.at[k], sem_i)

    def wait_idx():
        pltpu.make_async_copy(edges_hbm.at[pl.ds(0, 2), pl.ds(0, CHUNK)],
                              ring.at[0], sem_i).wait()

    def rows_slot(j):
        return rows.at[pl.ds(pl.multiple_of((j & 1) * CHUNK, CHUNK), CHUNK)]

    def fire_gather(j):
        pltpu.async_copy(g_hbm.at[ring.at[j & (RING - 1), 0]],
                         rows_slot(j), sem_g)

    for j in range(IDX_AHEAD):
        fire_idx(j)

    zero16 = jnp.zeros((16,), jnp.float32)

    def zrow(i, carry):
        for k in range(D // 16):
            rows[i, pl.ds(k * 16, 16)] = zero16
        return carry

    lax.fori_loop(0, CHUNK, zrow, 0)
    for z in range(ROWS_PER_TILE // CHUNK):
        pltpu.async_copy(
            rows.at[pl.ds(0, CHUNK)],
            acc.at[pl.ds(s * ROWS_PER_TILE + z * CHUNK, CHUNK)], sem_g)
    for _ in range(3):
        wait_idx()
    for z in range(ROWS_PER_TILE // CHUNK):
        pltpu.make_async_copy(
            rows.at[pl.ds(0, CHUNK)],
            acc.at[pl.ds(0, CHUNK)], sem_g).wait()
    fire_gather(0)
    fire_gather(1)
    plsc.subcore_barrier()

    def body(j, carry):
        @pl.when(j < NCHUNK - 3)
        def _():
            wait_idx()

        pltpu.make_async_copy(g_hbm.at[ring.at[0, 0]], rows_slot(j),
                              sem_g).wait()

        @pl.when((j == NCHUNK - 1) & jnp.logical_not(full))
        def _():
            k = j & (RING - 1)
            for t in range(CHUNK // 16):
                ring[k, 1, pl.ds(t * 16, 16)] = (
                    jnp.arange(16, dtype=jnp.int32) + (N_NODES + t * 16))

        pltpu.sync_copy(rows_slot(j), acc.at[ring.at[j & (RING - 1), 1]],
                        add=True)

        @pl.when(j + 2 < NCHUNK)
        def _():
            fire_gather(j + 2)

        @pl.when(j + IDX_AHEAD < NCHUNK)
        def _():
            fire_idx(j + IDX_AHEAD)

        return carry

    lax.fori_loop(0, NCHUNK, body, 0)
    plsc.subcore_barrier()
    pltpu.sync_copy(acc.at[pl.ds(s * ROWS_PER_TILE, ROWS_PER_TILE)],
                    out_hbm.at[c, pl.ds(s * ROWS_PER_TILE, ROWS_PER_TILE)])


def _tc1_body(degpt_ref, x_ref, w1_ref, dis_ref, g1_ref):
    deg = jnp.sum(degpt_ref[...], axis=1, keepdims=True) + 1.0
    dis = lax.rsqrt(deg)
    dis_ref[...] = dis
    h = jnp.dot(x_ref[...], w1_ref[...],
                preferred_element_type=jnp.float32,
                precision=lax.Precision.DEFAULT)
    g1_ref[...] = h * dis


def _tc2_body(s1_ref, g1_ref, dis_ref, b1_ref, w2_ref, g2_ref):
    dis = dis_ref[...]
    ssum = (s1_ref[0] + s1_ref[1])[:N_NODES]
    t = (ssum + g1_ref[...]) * dis + b1_ref[...]
    t = jnp.maximum(t, 0.0)
    h2 = jnp.dot(t, w2_ref[...],
                 preferred_element_type=jnp.float32,
                 precision=lax.Precision.DEFAULT)
    g2_ref[...] = h2 * dis


def _tc3_body(s2_ref, g2_ref, dis_ref, b2_ref, out_ref):
    ssum = (s2_ref[0] + s2_ref[1])[:N_NODES]
    out_ref[...] = (ssum + g2_ref[...]) * dis_ref[...] + b2_ref[...]


_tc1 = pl.pallas_call(
    _tc1_body,
    out_shape=[jax.ShapeDtypeStruct((N_NODES, 1), jnp.float32),
               jax.ShapeDtypeStruct((N_NODES, D), jnp.float32)],
)

_tc2 = pl.pallas_call(
    _tc2_body,
    out_shape=jax.ShapeDtypeStruct((N_NODES, D), jnp.float32),
)

_tc3 = pl.pallas_call(
    _tc3_body,
    out_shape=jax.ShapeDtypeStruct((N_NODES, D), jnp.float32),
)


def kernel(x, edge_index, W1, b1, W2, b2):
    edges = edge_index.astype(jnp.int32)
    b1r = b1.reshape(1, D)
    b2r = b2.reshape(1, D)

    degp = _deg_kernel(edges)
    dis, g1 = _tc1(degp.T, x, W1)
    s1 = _scatter_kernel(edges, g1)
    g2 = _tc2(s1, g1, dis, b1r, W2)
    s2 = _scatter_kernel(edges, g2)
    return _tc3(s2, g2, dis, b2r)


# speedup vs baseline: 2.2871x; 1.0224x over previous
import functools

import jax
import jax.numpy as jnp
from jax import lax
from jax.experimental import pallas as pl
from jax.experimental.pallas import tpu as pltpu
from jax.experimental.pallas import tpu_sc as plsc

N_NODES = 10000
D = 128
N_EDGES = 320000
NC = 2
NS = 16
NW = NC * NS
CHUNK = 128
TOTAL_CHUNKS = N_EDGES // CHUNK
BASE_CHUNKS = TOTAL_CHUNKS // NW
EXTRA = TOTAL_CHUNKS - NW * BASE_CHUNKS
NCHUNK = BASE_CHUNKS + 1
N_PAD = 10240
ROWS_PER_TILE = N_PAD // NS
RING = 8
IDX_AHEAD = 6

_MESH = plsc.VectorSubcoreMesh(core_axis_name="c", subcore_axis_name="s")


@functools.partial(
    pl.kernel,
    out_type=jax.ShapeDtypeStruct((NW, N_NODES), jnp.float32),
    mesh=_MESH,
    scratch_types=[
        pltpu.VMEM((2, NCHUNK * CHUNK), jnp.int32),
        pltpu.VMEM((N_NODES,), jnp.float32),
    ],
    compiler_params=pltpu.CompilerParams(needs_layout_passes=False),
)
def _deg_kernel(edges_hbm, out_hbm, idx_d, degp):
    c = lax.axis_index("c")
    s = lax.axis_index("s")
    w = c * NS + s

    def zero(i, carry):
        degp[pl.ds(i * 16, 16)] = jnp.zeros((16,), jnp.float32)
        return carry

    lax.fori_loop(0, N_NODES // 16, zero, 0)

    c0 = w * BASE_CHUNKS + jnp.minimum(w, EXTRA)
    full = w < EXTRA
    off = pl.multiple_of(c0 * CHUNK, CHUNK)

    @pl.when(full)
    def _():
        pltpu.sync_copy(
            edges_hbm.at[pl.ds(0, 2), pl.ds(off, NCHUNK * CHUNK)], idx_d)

    @pl.when(jnp.logical_not(full))
    def _():
        pltpu.sync_copy(
            edges_hbm.at[pl.ds(0, 2), pl.ds(off, BASE_CHUNKS * CHUNK)],
            idx_d.at[pl.ds(0, 2), pl.ds(0, BASE_CHUNKS * CHUNK)])

    ones = jnp.ones((16,), jnp.float32)

    def body(i, carry):
        idx16 = idx_d[1, pl.ds(i * 16, 16)]
        plsc.addupdate_scatter(degp, [idx16], ones)
        return carry

    lax.fori_loop(0, BASE_CHUNKS * CHUNK // 16, body, 0)

    @pl.when(full)
    def _():
        lax.fori_loop(BASE_CHUNKS * CHUNK // 16, NCHUNK * CHUNK // 16,
                      body, 0)

    pltpu.sync_copy(degp, out_hbm.at[w])


@functools.partial(
    pl.kernel,
    out_type=jax.ShapeDtypeStruct((NC, N_PAD, D), jnp.float32),
    mesh=_MESH,
    scratch_types=[
        pltpu.VMEM((RING, 2, CHUNK), jnp.int32),
        pltpu.VMEM((2 * CHUNK, D), jnp.float32),
        pltpu.VMEM_SHARED((N_PAD, D), jnp.float32),
        pltpu.SemaphoreType.DMA,
        pltpu.SemaphoreType.DMA,
    ],
)
def _scatter_kernel(edges_hbm, g_hbm, out_hbm,
                    ring, rows, acc, sem_i, sem_g):
    c = lax.axis_index("c")
    s = lax.axis_index("s")
    w = c * NS + s
    c0 = w * BASE_CHUNKS + jnp.minimum(w, EXTRA)
    full = w < EXTRA

    def fire_idx(j):
        k = j & (RING - 1)
        g = jnp.minimum(c0 + j, TOTAL_CHUNKS - 1)
        off = pl.multiple_of(g * CHUNK, CHUNK)
        pltpu.async_copy(edges_hbm.at[pl.ds(0, 2), pl.ds(off, CHUNK)],
                         ring.at[k], sem_i)

    def wait_idx():
        pltpu.make_async_copy(edges_hbm.at[pl.ds(0, 2), pl.ds(0, CHUNK)],
                              ring.at[0], sem_i).wait()

    def rows_slot(j):
        return rows.at[pl.ds(pl.multiple_of((j & 1) * CHUNK, CHUNK), CHUNK)]

    def fire_gather(j):
        pltpu.async_copy(g_hbm.at[ring.at[j & (RING - 1), 0]],
                         rows_slot(j), sem_g)

    for j in range(IDX_AHEAD):
        fire_idx(j)

    zero16 = jnp.zeros((16,), jnp.float32)

    def zrow(i, carry):
        for k in range(D // 16):
            rows[i, pl.ds(k * 16, 16)] = zero16
        return carry

    lax.fori_loop(0, CHUNK, zrow, 0)
    for z in range(ROWS_PER_TILE // CHUNK):
        pltpu.async_copy(
            rows.at[pl.ds(0, CHUNK)],
            acc.at[pl.ds(s * ROWS_PER_TILE + z * CHUNK, CHUNK)], sem_g)
    for _ in range(3):
        wait_idx()
    for z in range(ROWS_PER_TILE // CHUNK):
        pltpu.make_async_copy(
            rows.at[pl.ds(0, CHUNK)],
            acc.at[pl.ds(0, CHUNK)], sem_g).wait()
    fire_gather(0)
    fire_gather(1)
    plsc.subcore_barrier()

    def body(j, carry):
        @pl.when(j < NCHUNK - 3)
        def _():
            wait_idx()

        pltpu.make_async_copy(g_hbm.at[ring.at[0, 0]], rows_slot(j),
                              sem_g).wait()

        @pl.when((j == NCHUNK - 1) & jnp.logical_not(full))
        def _():
            k = j & (RING - 1)
            for t in range(CHUNK // 16):
                ring[k, 1, pl.ds(t * 16, 16)] = (
                    jnp.arange(16, dtype=jnp.int32) + (N_NODES + t * 16))

        pltpu.sync_copy(rows_slot(j), acc.at[ring.at[j & (RING - 1), 1]],
                        add=True)

        @pl.when(j + 2 < NCHUNK)
        def _():
            fire_gather(j + 2)

        @pl.when(j + IDX_AHEAD < NCHUNK)
        def _():
            fire_idx(j + IDX_AHEAD)

        return carry

    lax.fori_loop(0, NCHUNK, body, 0)
    plsc.subcore_barrier()
    pltpu.sync_copy(acc.at[pl.ds(s * ROWS_PER_TILE, ROWS_PER_TILE)],
                    out_hbm.at[c, pl.ds(s * ROWS_PER_TILE, ROWS_PER_TILE)])


def _tc1_body(degp_ref, x_ref, w1_ref, dis_ref, g1_ref):
    deg = jnp.sum(degp_ref[...], axis=0)[:, None] + 1.0
    dis = lax.rsqrt(deg)
    dis_ref[...] = dis
    h = jnp.dot(x_ref[...], w1_ref[...],
                preferred_element_type=jnp.float32,
                precision=lax.Precision.DEFAULT)
    g1_ref[...] = h * dis


def _tc2_body(s1_ref, g1_ref, dis_ref, b1_ref, w2_ref, g2_ref):
    dis = dis_ref[...]
    ssum = (s1_ref[0] + s1_ref[1])[:N_NODES]
    t = (ssum + g1_ref[...]) * dis + b1_ref[...]
    t = jnp.maximum(t, 0.0)
    h2 = jnp.dot(t, w2_ref[...],
                 preferred_element_type=jnp.float32,
                 precision=lax.Precision.DEFAULT)
    g2_ref[...] = h2 * dis


def _tc3_body(s2_ref, g2_ref, dis_ref, b2_ref, out_ref):
    ssum = (s2_ref[0] + s2_ref[1])[:N_NODES]
    out_ref[...] = (ssum + g2_ref[...]) * dis_ref[...] + b2_ref[...]


_tc1 = pl.pallas_call(
    _tc1_body,
    out_shape=[jax.ShapeDtypeStruct((N_NODES, 1), jnp.float32),
               jax.ShapeDtypeStruct((N_NODES, D), jnp.float32)],
)

_tc2 = pl.pallas_call(
    _tc2_body,
    out_shape=jax.ShapeDtypeStruct((N_NODES, D), jnp.float32),
)

_tc3 = pl.pallas_call(
    _tc3_body,
    out_shape=jax.ShapeDtypeStruct((N_NODES, D), jnp.float32),
)


def kernel(x, edge_index, W1, b1, W2, b2):
    edges = edge_index.astype(jnp.int32)
    b1r = b1.reshape(1, D)
    b2r = b2.reshape(1, D)

    degp = _deg_kernel(edges)
    dis, g1 = _tc1(degp, x, W1)
    s1 = _scatter_kernel(edges, g1)
    g2 = _tc2(s1, g1, dis, b1r, W2)
    s2 = _scatter_kernel(edges, g2)
    return _tc3(s2, g2, dis, b2r)
